# baseline (speedup 1.0000x reference)
import functools

import jax
import jax.numpy as jnp
from jax import lax
from jax.experimental import pallas as pl
from jax.experimental.pallas import tpu as pltpu
from jax.experimental.pallas import tpu_sc as plsc

LANES = 16


def _sc_gather_kernel(B, L, nv, bpw, ncats, mdim):
  mesh = plsc.VectorSubcoreMesh(core_axis_name="c", subcore_axis_name="s")
  num_cores = mesh.num_cores

  @functools.partial(
      pl.kernel,
      out_type=(
          jax.ShapeDtypeStruct((mdim, B), jnp.float32),
          jax.ShapeDtypeStruct((16, B), jnp.float32),
          jax.ShapeDtypeStruct((B,), jnp.float32),
      ),
      mesh=mesh,
      compiler_params=pltpu.CompilerParams(needs_layout_passes=False),
      scratch_types=[
          pltpu.VMEM((bpw,), jnp.int32),
          pltpu.VMEM((bpw,), jnp.int32),
          pltpu.VMEM((bpw, 128), jnp.float32),
          pltpu.VMEM((mdim, bpw), jnp.float32),
          pltpu.VMEM((bpw,), jnp.float32),
          pltpu.VMEM((L, bpw), jnp.int32),
          pltpu.VMEM((ncats * 16,), jnp.float32),
          pltpu.VMEM((16, bpw), jnp.float32),
          pltpu.SemaphoreType.DMA,
          pltpu.SemaphoreType.DMA,
      ],
  )
  def body(mid_hbm, cats_hbm, movies4_hbm, cattab_hbm, bias_hbm,
           rows_out, mean_out, bias_out,
           idx_v, idx4_v, rows128_v, rows_t_v, bias_v, cats_v, tab_v, mean_v,
           sem_r, sem_b):
    wid = lax.axis_index("s") * num_cores + lax.axis_index("c")
    base = wid * bpw
    pltpu.sync_copy(mid_hbm.at[pl.ds(base, bpw)], idx_v)

    def shift_grp(g, carry):
      b0 = g * LANES
      idx4_v[pl.ds(b0, LANES)] = lax.shift_right_logical(
          idx_v[pl.ds(b0, LANES)], 2)
      return carry
    lax.fori_loop(0, bpw // LANES, shift_grp, 0)
    cp_rows = pltpu.async_copy(movies4_hbm.at[idx4_v], rows128_v, sem_r)
    cp_bias = pltpu.async_copy(bias_hbm.at[idx_v], bias_v, sem_b)

    pltpu.sync_copy(cats_hbm.at[:, pl.ds(base, bpw)], cats_v)
    pltpu.sync_copy(cattab_hbm, tab_v)

    lane = lax.iota(jnp.int32, LANES)

    def group(g, carry):
      b0 = g * LANES
      cnt = jnp.zeros((LANES,), jnp.float32)
      acc = [jnp.zeros((LANES,), jnp.float32) for _ in range(16)]
      for l in range(L):
        idxs = cats_v[l, pl.ds(b0, LANES)]
        cnt = cnt + jnp.where(idxs != 0, 1.0, 0.0)
        flat = idxs * 16
        for d in range(16):
          acc[d] = acc[d] + plsc.load_gather(tab_v, [flat + d])
      inv = jnp.where(cnt > 0, 1.0 / jnp.maximum(cnt, 1.0), 0.0)
      for d in range(16):
        mean_v[d, pl.ds(b0, LANES)] = acc[d] * inv
      return carry

    lax.fori_loop(0, bpw // LANES, group, 0)
    pltpu.sync_copy(mean_v, mean_out.at[:, pl.ds(base, bpw)])

    cp_rows.wait()

    def extract_grp(g, carry):
      b0 = g * LANES
      blane = b0 + lane
      off = jnp.bitwise_and(idx_v[pl.ds(b0, LANES)], 3) * mdim
      for d in range(mdim):
        rows_t_v[d, pl.ds(b0, LANES)] = plsc.load_gather(
            rows128_v, [blane, off + d])
      return carry

    lax.fori_loop(0, bpw // LANES, extract_grp, 0)
    pltpu.sync_copy(rows_t_v, rows_out.at[:, pl.ds(base, bpw)])

    cp_bias.wait()
    pltpu.sync_copy(bias_v, bias_out.at[pl.ds(base, bpw)])

  return body


def _tr_body(c_ref, o_ref):
  o_ref[...] = c_ref[...].T


def _fc_body(rows_t_ref, mean_t_ref, w1_ref, w2_ref, b_ref, out_ref):
  a = jnp.maximum(rows_t_ref[...], 0.0)
  c = jnp.maximum(mean_t_ref[...], 0.0)
  dn = (((0,), (0,)), ((), ()))
  out_ref[...] = (
      lax.dot_general(a, w1_ref[...], dn, preferred_element_type=jnp.float32)
      + lax.dot_general(c, w2_ref[...], dn, preferred_element_type=jnp.float32)
      + b_ref[...]
  )


def kernel(movie_id, movie_categories, emb_movies, emb_cats, bias_movie,
           fc_w, fc_b):
  B = movie_id.shape[0]
  L = movie_categories.shape[1]
  ncats, cdim = emb_cats.shape
  mdim = emb_movies.shape[1]
  assert cdim == 16 and mdim == 32

  info = plsc.get_sparse_core_info()
  nv = info.num_cores * info.num_subcores
  bpw = B // nv

  mid = movie_id.astype(jnp.int32)
  tab_flat = emb_cats.reshape(-1)
  movies4 = emb_movies.reshape(-1, 128)
  bias_flat = bias_movie.reshape(-1)

  TRB = 2048
  cats_t = pl.pallas_call(
      _tr_body,
      grid=(B // TRB,),
      in_specs=[pl.BlockSpec((TRB, L), lambda i: (i, 0))],
      out_specs=pl.BlockSpec((L, TRB), lambda i: (0, i)),
      out_shape=jax.ShapeDtypeStruct((L, B), jnp.int32),
  )(movie_categories.astype(jnp.int32))

  sc = _sc_gather_kernel(B, L, nv, bpw, ncats, mdim)
  rows_t, mean_t, bias = sc(mid, cats_t, movies4, tab_flat, bias_flat)

  w1 = fc_w.T[:mdim]
  w2 = fc_w.T[mdim:]
  out_dim = fc_w.shape[0]

  movie_vec = pl.pallas_call(
      _fc_body,
      out_shape=jax.ShapeDtypeStruct((B, out_dim), jnp.float32),
  )(rows_t, mean_t, w1, w2, fc_b.reshape(1, out_dim))

  return movie_vec, bias


# speedup vs baseline: 1.4060x; 1.4060x over previous
import functools

import jax
import jax.numpy as jnp
from jax import lax
from jax.experimental import pallas as pl
from jax.experimental.pallas import tpu as pltpu
from jax.experimental.pallas import tpu_sc as plsc

LANES = 16


def _sc_gather_kernel(B, L, nv, bpw, ncats, mdim):
  mesh = plsc.VectorSubcoreMesh(core_axis_name="c", subcore_axis_name="s")
  num_cores = mesh.num_cores

  @functools.partial(
      pl.kernel,
      out_type=(
          jax.ShapeDtypeStruct((mdim, B), jnp.float32),
          jax.ShapeDtypeStruct((16, B), jnp.float32),
          jax.ShapeDtypeStruct((B,), jnp.float32),
      ),
      mesh=mesh,
      compiler_params=pltpu.CompilerParams(needs_layout_passes=False),
      scratch_types=[
          pltpu.VMEM((bpw,), jnp.int32),
          pltpu.VMEM((bpw,), jnp.int32),
          pltpu.VMEM((bpw, 128), jnp.float32),
          pltpu.VMEM((mdim, bpw), jnp.float32),
          pltpu.VMEM((bpw,), jnp.float32),
          pltpu.VMEM((L, bpw), jnp.int32),
          pltpu.VMEM((ncats * 16,), jnp.float32),
          pltpu.VMEM((16, bpw), jnp.float32),
          pltpu.SemaphoreType.DMA,
          pltpu.SemaphoreType.DMA,
      ],
  )
  def body(mid_hbm, cats_hbm, movies4_hbm, cattab_hbm, bias_hbm,
           rows_out, mean_out, bias_out,
           idx_v, idx4_v, rows128_v, rows_t_v, bias_v, cats_v, tab_v, mean_v,
           sem_r, sem_b):
    wid = lax.axis_index("s") * num_cores + lax.axis_index("c")
    base = wid * bpw
    pltpu.sync_copy(mid_hbm.at[pl.ds(base, bpw)], idx_v)

    def shift_grp(g, carry):
      b0 = g * LANES
      idx4_v[pl.ds(b0, LANES)] = jnp.bitwise_and(
          idx_v[pl.ds(b0, LANES)], 0x3FFFF)
      return carry
    lax.fori_loop(0, bpw // LANES, shift_grp, 0)
    cp_rows = pltpu.async_copy(movies4_hbm.at[idx4_v], rows128_v, sem_r)
    cp_bias = pltpu.async_copy(bias_hbm.at[idx_v], bias_v, sem_b)

    pltpu.sync_copy(cats_hbm.at[:, pl.ds(base, bpw)], cats_v)
    pltpu.sync_copy(cattab_hbm, tab_v)

    lane = lax.iota(jnp.int32, LANES)

    def group(g, carry):
      b0 = g * LANES
      cnt = jnp.zeros((LANES,), jnp.float32)
      acc = [jnp.zeros((LANES,), jnp.float32) for _ in range(16)]
      for l in range(L):
        idxs = cats_v[l, pl.ds(b0, LANES)]
        cnt = cnt + jnp.where(idxs != 0, 1.0, 0.0)
        flat = idxs * 16
        for d in range(16):
          acc[d] = acc[d] + plsc.load_gather(tab_v, [flat + d])
      inv = jnp.where(cnt > 0, 1.0 / jnp.maximum(cnt, 1.0), 0.0)
      for d in range(16):
        mean_v[d, pl.ds(b0, LANES)] = acc[d] * inv
      return carry

    lax.fori_loop(0, bpw // LANES, group, 0)
    pltpu.sync_copy(mean_v, mean_out.at[:, pl.ds(base, bpw)])

    cp_rows.wait()

    def extract_grp(g, carry):
      b0 = g * LANES
      blane = b0 + lane
      off = lax.shift_right_logical(idx_v[pl.ds(b0, LANES)], 18) * mdim
      for d in range(mdim):
        rows_t_v[d, pl.ds(b0, LANES)] = plsc.load_gather(
            rows128_v, [blane, off + d])
      return carry

    lax.fori_loop(0, bpw // LANES, extract_grp, 0)
    pltpu.sync_copy(rows_t_v, rows_out.at[:, pl.ds(base, bpw)])

    cp_bias.wait()
    pltpu.sync_copy(bias_v, bias_out.at[pl.ds(base, bpw)])

  return body


def _tr_body(c_ref, o_ref):
  o_ref[...] = c_ref[...].T


def _pack_body(t0_ref, t1_ref, t2_ref, t3_ref, eye_ref, o_ref):
  dn = (((0,), (0,)), ((), ()))
  eye = eye_ref[...]
  o_ref[...] = jnp.concatenate(
      [lax.dot_general(t_ref[...], eye, dn, preferred_element_type=jnp.float32)
       for t_ref in (t0_ref, t1_ref, t2_ref, t3_ref)], axis=1)


def _fc_body(rows_t_ref, mean_t_ref, w1_ref, w2_ref, b_ref, out_ref):
  a = jnp.maximum(rows_t_ref[...], 0.0)
  c = jnp.maximum(mean_t_ref[...], 0.0)
  dn = (((0,), (0,)), ((), ()))
  out_ref[...] = (
      lax.dot_general(a, w1_ref[...], dn, preferred_element_type=jnp.float32)
      + lax.dot_general(c, w2_ref[...], dn, preferred_element_type=jnp.float32)
      + b_ref[...]
  )


def kernel(movie_id, movie_categories, emb_movies, emb_cats, bias_movie,
           fc_w, fc_b):
  B = movie_id.shape[0]
  L = movie_categories.shape[1]
  ncats, cdim = emb_cats.shape
  mdim = emb_movies.shape[1]
  assert cdim == 16 and mdim == 32

  info = plsc.get_sparse_core_info()
  nv = info.num_cores * info.num_subcores
  bpw = B // nv

  mid = movie_id.astype(jnp.int32)
  tab_flat = emb_cats.reshape(-1)
  bias_flat = bias_movie.reshape(-1)

  nmov = emb_movies.shape[0]
  STRIDE = 1 << 18
  PBLK = 2048
  nblk = STRIDE // PBLK
  movies_t = emb_movies.T
  last_blk = (nmov - 1) // PBLK

  def _in_spec(m):
    return pl.BlockSpec(
        (mdim, PBLK),
        lambda i, m=m: (0, jnp.minimum(i + m * nblk, last_blk)))

  movies4 = pl.pallas_call(
      _pack_body,
      grid=(nblk,),
      in_specs=[_in_spec(0), _in_spec(1), _in_spec(2), _in_spec(3),
                pl.BlockSpec((mdim, mdim), lambda i: (0, 0))],
      out_specs=pl.BlockSpec((PBLK, 128), lambda i: (i, 0)),
      out_shape=jax.ShapeDtypeStruct((STRIDE, 128), jnp.float32),
      compiler_params=pltpu.CompilerParams(fuse_transposed_lhs_in_matmul=True),
  )(movies_t, movies_t, movies_t, movies_t, jnp.eye(mdim, dtype=jnp.float32))

  TRB = 2048
  cats_t = pl.pallas_call(
      _tr_body,
      grid=(B // TRB,),
      in_specs=[pl.BlockSpec((TRB, L), lambda i: (i, 0))],
      out_specs=pl.BlockSpec((L, TRB), lambda i: (0, i)),
      out_shape=jax.ShapeDtypeStruct((L, B), jnp.int32),
  )(movie_categories.astype(jnp.int32))

  sc = _sc_gather_kernel(B, L, nv, bpw, ncats, mdim)
  rows_t, mean_t, bias = sc(mid, cats_t, movies4, tab_flat, bias_flat)

  w1 = fc_w.T[:mdim]
  w2 = fc_w.T[mdim:]
  out_dim = fc_w.shape[0]

  movie_vec = pl.pallas_call(
      _fc_body,
      out_shape=jax.ShapeDtypeStruct((B, out_dim), jnp.float32),
      compiler_params=pltpu.CompilerParams(fuse_transposed_lhs_in_matmul=True),
  )(rows_t, mean_t, w1, w2, fc_b.reshape(1, out_dim))

  return movie_vec, bias


# speedup vs baseline: 1.4540x; 1.0341x over previous
import functools

import jax
import jax.numpy as jnp
from jax import lax
from jax.experimental import pallas as pl
from jax.experimental.pallas import tpu as pltpu
from jax.experimental.pallas import tpu_sc as plsc

LANES = 16


def _sc_bag_kernel(B, L, nv, bpw, ncats):
  mesh = plsc.VectorSubcoreMesh(core_axis_name="c", subcore_axis_name="s")
  num_cores = mesh.num_cores

  @functools.partial(
      pl.kernel,
      out_type=(
          jax.ShapeDtypeStruct((16, B), jnp.float32),
          jax.ShapeDtypeStruct((B,), jnp.float32),
      ),
      mesh=mesh,
      compiler_params=pltpu.CompilerParams(needs_layout_passes=False),
      scratch_types=[
          pltpu.VMEM((bpw,), jnp.int32),
          pltpu.VMEM((bpw,), jnp.float32),
          pltpu.VMEM((L, bpw), jnp.int32),
          pltpu.VMEM((ncats * 16,), jnp.float32),
          pltpu.VMEM((16, bpw), jnp.float32),
          pltpu.SemaphoreType.DMA,
      ],
  )
  def body(mid_hbm, cats_hbm, cattab_hbm, bias_hbm,
           mean_out, bias_out,
           idx_v, bias_v, cats_v, tab_v, mean_v, sem_b):
    wid = lax.axis_index("s") * num_cores + lax.axis_index("c")
    base = wid * bpw
    pltpu.sync_copy(mid_hbm.at[pl.ds(base, bpw)], idx_v)
    cp_bias = pltpu.async_copy(bias_hbm.at[idx_v], bias_v, sem_b)

    pltpu.sync_copy(cats_hbm.at[:, pl.ds(base, bpw)], cats_v)
    pltpu.sync_copy(cattab_hbm, tab_v)

    def group(g, carry):
      b0 = g * LANES
      cnt = jnp.zeros((LANES,), jnp.float32)
      acc = [jnp.zeros((LANES,), jnp.float32) for _ in range(16)]
      for l in range(L):
        idxs = cats_v[l, pl.ds(b0, LANES)]
        cnt = cnt + jnp.where(idxs != 0, 1.0, 0.0)
        flat = idxs * 16
        for d in range(16):
          acc[d] = acc[d] + plsc.load_gather(tab_v, [flat + d])
      inv = jnp.where(cnt > 0, 1.0 / jnp.maximum(cnt, 1.0), 0.0)
      for d in range(16):
        mean_v[d, pl.ds(b0, LANES)] = acc[d] * inv
      return carry

    lax.fori_loop(0, bpw // LANES, group, 0)
    pltpu.sync_copy(mean_v, mean_out.at[:, pl.ds(base, bpw)])

    cp_bias.wait()
    pltpu.sync_copy(bias_v, bias_out.at[pl.ds(base, bpw)])

  return body


def _sc_rows_kernel(B, nv, bpw, mdim):
  mesh = plsc.VectorSubcoreMesh(core_axis_name="c", subcore_axis_name="s")
  num_cores = mesh.num_cores

  @functools.partial(
      pl.kernel,
      out_type=jax.ShapeDtypeStruct((mdim, B), jnp.float32),
      mesh=mesh,
      compiler_params=pltpu.CompilerParams(needs_layout_passes=False),
      scratch_types=[
          pltpu.VMEM((bpw,), jnp.int32),
          pltpu.VMEM((bpw,), jnp.int32),
          pltpu.VMEM((bpw, 128), jnp.float32),
          pltpu.VMEM((mdim, bpw), jnp.float32),
          pltpu.SemaphoreType.DMA,
      ],
  )
  def body(mid_hbm, movies4_hbm, rows_out,
           idx_v, idx4_v, rows128_v, rows_t_v, sem_r):
    wid = lax.axis_index("s") * num_cores + lax.axis_index("c")
    base = wid * bpw
    pltpu.sync_copy(mid_hbm.at[pl.ds(base, bpw)], idx_v)

    def shift_grp(g, carry):
      b0 = g * LANES
      idx4_v[pl.ds(b0, LANES)] = jnp.bitwise_and(
          idx_v[pl.ds(b0, LANES)], 0x3FFFF)
      return carry
    lax.fori_loop(0, bpw // LANES, shift_grp, 0)
    cp_rows = pltpu.async_copy(movies4_hbm.at[idx4_v], rows128_v, sem_r)

    lane = lax.iota(jnp.int32, LANES)
    cp_rows.wait()

    def extract_grp(g, carry):
      b0 = g * LANES
      blane = b0 + lane
      off = lax.shift_right_logical(idx_v[pl.ds(b0, LANES)], 18) * mdim
      for d in range(mdim):
        rows_t_v[d, pl.ds(b0, LANES)] = plsc.load_gather(
            rows128_v, [blane, off + d])
      return carry

    lax.fori_loop(0, bpw // LANES, extract_grp, 0)
    pltpu.sync_copy(rows_t_v, rows_out.at[:, pl.ds(base, bpw)])

  return body


def _tr_body(c_ref, o_ref):
  o_ref[...] = c_ref[...].T


def _pack_body(t0_ref, t1_ref, t2_ref, t3_ref, eye_ref, o_ref):
  dn = (((0,), (0,)), ((), ()))
  eye = eye_ref[...]
  o_ref[...] = jnp.concatenate(
      [lax.dot_general(t_ref[...], eye, dn, preferred_element_type=jnp.float32)
       for t_ref in (t0_ref, t1_ref, t2_ref, t3_ref)], axis=1)


def _fc_body(rows_t_ref, mean_t_ref, w1_ref, w2_ref, b_ref, out_ref):
  a = jnp.maximum(rows_t_ref[...], 0.0)
  c = jnp.maximum(mean_t_ref[...], 0.0)
  dn = (((0,), (0,)), ((), ()))
  out_ref[...] = (
      lax.dot_general(a, w1_ref[...], dn, preferred_element_type=jnp.float32)
      + lax.dot_general(c, w2_ref[...], dn, preferred_element_type=jnp.float32)
      + b_ref[...]
  )


def kernel(movie_id, movie_categories, emb_movies, emb_cats, bias_movie,
           fc_w, fc_b):
  B = movie_id.shape[0]
  L = movie_categories.shape[1]
  ncats, cdim = emb_cats.shape
  mdim = emb_movies.shape[1]
  assert cdim == 16 and mdim == 32

  info = plsc.get_sparse_core_info()
  nv = info.num_cores * info.num_subcores
  bpw = B // nv

  mid = movie_id.astype(jnp.int32)
  tab_flat = emb_cats.reshape(-1)

  nmov = emb_movies.shape[0]
  STRIDE = 1 << 18
  PBLK = 2048
  BBLK = 8192
  nblk = STRIDE // PBLK
  movies_t = emb_movies.T
  last_blk = (nmov - 1) // PBLK
  last_bblk = (nmov - 1) // BBLK

  def _in_spec(m):
    return pl.BlockSpec(
        (mdim, PBLK),
        lambda i, m=m: (0, jnp.minimum(i + m * nblk, last_blk)))

  movies4 = pl.pallas_call(
      _pack_body,
      grid=(nblk,),
      in_specs=[_in_spec(0), _in_spec(1), _in_spec(2), _in_spec(3),
                pl.BlockSpec((mdim, mdim), lambda i: (0, 0))],
      out_specs=pl.BlockSpec((PBLK, 128), lambda i: (i, 0)),
      out_shape=jax.ShapeDtypeStruct((STRIDE, 128), jnp.float32),
      compiler_params=pltpu.CompilerParams(fuse_transposed_lhs_in_matmul=True),
  )(movies_t, movies_t, movies_t, movies_t, jnp.eye(mdim, dtype=jnp.float32))

  bias_flat = bias_movie.reshape(-1)

  TRB = 2048
  cats_t = pl.pallas_call(
      _tr_body,
      grid=(B // TRB,),
      in_specs=[pl.BlockSpec((TRB, L), lambda i: (i, 0))],
      out_specs=pl.BlockSpec((L, TRB), lambda i: (0, i)),
      out_shape=jax.ShapeDtypeStruct((L, B), jnp.int32),
  )(movie_categories.astype(jnp.int32))

  sc_bag = _sc_bag_kernel(B, L, nv, bpw, ncats)
  mean_t, bias = sc_bag(mid, cats_t, tab_flat, bias_flat)

  sc_rows = _sc_rows_kernel(B, nv, bpw, mdim)
  rows_t = sc_rows(mid, movies4)

  w1 = fc_w.T[:mdim]
  w2 = fc_w.T[mdim:]
  out_dim = fc_w.shape[0]

  movie_vec = pl.pallas_call(
      _fc_body,
      out_shape=jax.ShapeDtypeStruct((B, out_dim), jnp.float32),
      compiler_params=pltpu.CompilerParams(fuse_transposed_lhs_in_matmul=True),
  )(rows_t, mean_t, w1, w2, fc_b.reshape(1, out_dim))

  return movie_vec, bias


# speedup vs baseline: 2.1328x; 1.4668x over previous
import functools

import jax
import jax.numpy as jnp
from jax import lax
from jax.experimental import pallas as pl
from jax.experimental.pallas import tpu as pltpu
from jax.experimental.pallas import tpu_sc as plsc

LANES = 16


def _sc_bag_kernel(B, L, nv, bpw, ncats):
  mesh = plsc.VectorSubcoreMesh(core_axis_name="c", subcore_axis_name="s")
  num_cores = mesh.num_cores

  @functools.partial(
      pl.kernel,
      out_type=(
          jax.ShapeDtypeStruct((16, B), jnp.float32),
          jax.ShapeDtypeStruct((B,), jnp.float32),
      ),
      mesh=mesh,
      compiler_params=pltpu.CompilerParams(needs_layout_passes=False),
      scratch_types=[
          pltpu.VMEM((bpw,), jnp.int32),
          pltpu.VMEM((bpw,), jnp.float32),
          pltpu.VMEM((L, bpw), jnp.int32),
          pltpu.VMEM((ncats * 16,), jnp.float32),
          pltpu.VMEM((16, bpw), jnp.float32),
          pltpu.SemaphoreType.DMA,
      ],
  )
  def body(mid_hbm, cats_hbm, cattab_hbm, bias_hbm,
           mean_out, bias_out,
           idx_v, bias_v, cats_v, tab_v, mean_v, sem_b):
    wid = lax.axis_index("s") * num_cores + lax.axis_index("c")
    base = wid * bpw
    pltpu.sync_copy(mid_hbm.at[pl.ds(base, bpw)], idx_v)
    cp_bias = pltpu.async_copy(bias_hbm.at[idx_v], bias_v, sem_b)

    pltpu.sync_copy(cats_hbm.at[:, pl.ds(base, bpw)], cats_v)
    pltpu.sync_copy(cattab_hbm, tab_v)

    def group(g, carry):
      b0 = g * LANES
      cnt = jnp.zeros((LANES,), jnp.float32)
      acc = [jnp.zeros((LANES,), jnp.float32) for _ in range(16)]
      for l in range(L):
        idxs = cats_v[l, pl.ds(b0, LANES)]
        cnt = cnt + jnp.where(idxs != 0, 1.0, 0.0)
        flat = idxs * 16
        for d in range(16):
          acc[d] = acc[d] + plsc.load_gather(tab_v, [flat + d])
      inv = jnp.where(cnt > 0, 1.0 / jnp.maximum(cnt, 1.0), 0.0)
      for d in range(16):
        mean_v[d, pl.ds(b0, LANES)] = acc[d] * inv
      return carry

    lax.fori_loop(0, bpw // LANES, group, 0)
    pltpu.sync_copy(mean_v, mean_out.at[:, pl.ds(base, bpw)])

    cp_bias.wait()
    pltpu.sync_copy(bias_v, bias_out.at[pl.ds(base, bpw)])

  return body


def _sc_rows_kernel(B, nv, bpw, mdim):
  mesh = plsc.VectorSubcoreMesh(core_axis_name="c", subcore_axis_name="s")
  num_cores = mesh.num_cores

  @functools.partial(
      pl.kernel,
      out_type=jax.ShapeDtypeStruct((mdim, B), jnp.float32),
      mesh=mesh,
      compiler_params=pltpu.CompilerParams(needs_layout_passes=False),
      scratch_types=[
          pltpu.VMEM((bpw,), jnp.int32),
          pltpu.VMEM((bpw,), jnp.int32),
          pltpu.VMEM((bpw, 128), jnp.float32),
          pltpu.VMEM((mdim, bpw), jnp.float32),
          pltpu.SemaphoreType.DMA,
      ],
  )
  def body(mid_hbm, movies4_hbm, rows_out,
           idx_v, idx4_v, rows128_v, rows_t_v, sem_r):
    wid = lax.axis_index("s") * num_cores + lax.axis_index("c")
    base = wid * bpw
    pltpu.sync_copy(mid_hbm.at[pl.ds(base, bpw)], idx_v)

    def shift_grp(g, carry):
      b0 = g * LANES
      idx4_v[pl.ds(b0, LANES)] = jnp.bitwise_and(
          idx_v[pl.ds(b0, LANES)], 0x3FFFF)
      return carry
    lax.fori_loop(0, bpw // LANES, shift_grp, 0)
    cp_rows = pltpu.async_copy(movies4_hbm.at[idx4_v], rows128_v, sem_r)

    lane = lax.iota(jnp.int32, LANES)
    cp_rows.wait()

    def extract_grp(g, carry):
      b0 = g * LANES
      blane = b0 + lane
      off = lax.shift_right_logical(idx_v[pl.ds(b0, LANES)], 18) * mdim
      for d in range(mdim):
        rows_t_v[d, pl.ds(b0, LANES)] = plsc.load_gather(
            rows128_v, [blane, off + d])
      return carry

    lax.fori_loop(0, bpw // LANES, extract_grp, 0)
    pltpu.sync_copy(rows_t_v, rows_out.at[:, pl.ds(base, bpw)])

  return body


def _tr_body(c_ref, o_ref):
  o_ref[...] = c_ref[...].T


def _pack_body(t0_ref, t1_ref, t2_ref, t3_ref, o_ref):
  x = jnp.concatenate(
      [t0_ref[...], t1_ref[...], t2_ref[...], t3_ref[...]], axis=0)
  o_ref[...] = x.T


def _fc_body(rows_t_ref, mean_t_ref, w1_ref, w2_ref, b_ref, out_ref):
  a = jnp.maximum(rows_t_ref[...], 0.0)
  c = jnp.maximum(mean_t_ref[...], 0.0)
  dn = (((0,), (0,)), ((), ()))
  out_ref[...] = (
      lax.dot_general(a, w1_ref[...], dn, preferred_element_type=jnp.float32)
      + lax.dot_general(c, w2_ref[...], dn, preferred_element_type=jnp.float32)
      + b_ref[...]
  )


def kernel(movie_id, movie_categories, emb_movies, emb_cats, bias_movie,
           fc_w, fc_b):
  B = movie_id.shape[0]
  L = movie_categories.shape[1]
  ncats, cdim = emb_cats.shape
  mdim = emb_movies.shape[1]
  assert cdim == 16 and mdim == 32

  info = plsc.get_sparse_core_info()
  nv = info.num_cores * info.num_subcores
  bpw = B // nv

  mid = movie_id.astype(jnp.int32)
  tab_flat = emb_cats.reshape(-1)

  nmov = emb_movies.shape[0]
  STRIDE = 1 << 18
  PBLK = 2048
  BBLK = 8192
  nblk = STRIDE // PBLK
  movies_t = emb_movies.T
  last_blk = (nmov - 1) // PBLK
  last_bblk = (nmov - 1) // BBLK

  def _in_spec(m):
    return pl.BlockSpec(
        (mdim, PBLK),
        lambda i, m=m: (0, jnp.minimum(i + m * nblk, last_blk)))

  movies4 = pl.pallas_call(
      _pack_body,
      grid=(nblk,),
      in_specs=[_in_spec(0), _in_spec(1), _in_spec(2), _in_spec(3)],
      out_specs=pl.BlockSpec((PBLK, 128), lambda i: (i, 0)),
      out_shape=jax.ShapeDtypeStruct((STRIDE, 128), jnp.float32),
  )(movies_t, movies_t, movies_t, movies_t)

  bias_flat = bias_movie.reshape(-1)

  TRB = 2048
  cats_t = pl.pallas_call(
      _tr_body,
      grid=(B // TRB,),
      in_specs=[pl.BlockSpec((TRB, L), lambda i: (i, 0))],
      out_specs=pl.BlockSpec((L, TRB), lambda i: (0, i)),
      out_shape=jax.ShapeDtypeStruct((L, B), jnp.int32),
  )(movie_categories.astype(jnp.int32))

  sc_bag = _sc_bag_kernel(B, L, nv, bpw, ncats)
  mean_t, bias = sc_bag(mid, cats_t, tab_flat, bias_flat)

  sc_rows = _sc_rows_kernel(B, nv, bpw, mdim)
  rows_t = sc_rows(mid, movies4)

  w1 = fc_w.T[:mdim]
  w2 = fc_w.T[mdim:]
  out_dim = fc_w.shape[0]

  movie_vec = pl.pallas_call(
      _fc_body,
      out_shape=jax.ShapeDtypeStruct((B, out_dim), jnp.float32),
      compiler_params=pltpu.CompilerParams(fuse_transposed_lhs_in_matmul=True),
  )(rows_t, mean_t, w1, w2, fc_b.reshape(1, out_dim))

  return movie_vec, bias
